# baseline (speedup 1.0000x reference)
import functools

import jax
import jax.numpy as jnp
from jax import lax
from jax.experimental import pallas as pl
from jax.experimental.pallas import tpu as pltpu
from jax.experimental.pallas import tpu_sc as plsc

_B, _Q, _C = 32, 900, 91
_N = _Q * _C
_NPAD = 81920
_NV = _NPAD // 16
_K = 300
_KV = 19
_OUTP = 384
_OUTB = 1280
_NBINS = 1024
_PAD_IDX = _N + 4
_BOXPAD = 3712
_CAP = 4096

_i32 = jnp.int32
_f32 = jnp.float32


def _sc_body(probs_hbm, boxes_hbm, ts_hbm, sc_out, lb_out, bx_out,
             probs_v, boxes_v, ts_v, hist_v, suf_v, cand_v, gt_v, eq_v,
             val_v, key_v, key_a, pay_v, pay_a, outs_v, outl_v, outb_v, tsm):
    lanes = lax.broadcasted_iota(_i32, (16,), 0)
    wid = lax.axis_index("s") * 2 + lax.axis_index("c")

    pltpu.sync_copy(probs_hbm.at[wid], probs_v)
    pltpu.sync_copy(boxes_hbm.at[wid], boxes_v)
    pltpu.sync_copy(ts_hbm.at[wid], ts_v)

    suf_v[pl.ds(_NBINS, 16)] = jnp.zeros((16,), _i32)

    hist_lane_base = lanes * _NBINS
    ones = jnp.ones((16,), _i32)

    def popcnt(m):
        return plsc.all_reduce_population_count(m)[0]

    def zero_hist(nwords):
        def zb(i, _):
            hist_v[pl.ds(i * 16, 16)] = jnp.zeros((16,), _i32)
            return 0
        lax.fori_loop(0, nwords // 16, zb, 0)

    def suffix_pick(k_rem, lane_reduce):
        def sb(g_iter, carry):
            s_carry, dstar = carry
            g = 63 - g_iter
            if lane_reduce:
                tot = jnp.zeros((16,), _i32)
                for l in range(16):
                    tot = tot + hist_v[pl.ds(l * _NBINS + g * 16, 16)]
            else:
                tot = hist_v[pl.ds(g * 16, 16)]
            rcs = jnp.flip(plsc.cumsum(jnp.flip(tot))) + s_carry
            suf_v[pl.ds(g * 16, 16)] = rcs
            dvec = g * 16 + lanes
            cand = jnp.max(jnp.where(rcs >= k_rem, dvec, -1))
            return (jnp.max(rcs), jnp.maximum(dstar, cand))
        _, dstar = lax.fori_loop(0, 64, sb, (_i32(0), _i32(-1)))
        return dstar

    def suf_at(d):
        return jnp.max(plsc.load_gather(suf_v, [jnp.full((16,), 0, _i32) + d]))

    zero_hist(_NBINS * 16)

    def hist0_body(i, _):
        for u in range(4):
            v = probs_v[pl.ds(i * 64 + u * 16, 16)]
            bits = lax.bitcast_convert_type(v, _i32)
            m = bits >= 0
            digit = lax.shift_right_logical(bits, 20) & (_NBINS - 1)
            plsc.addupdate_scatter(hist_v, [hist_lane_base + digit], ones,
                                   mask=m)
        return 0
    lax.fori_loop(0, _NV // 4, hist0_body, 0)

    k_rem0 = _i32(_K)
    dstar0 = suffix_pick(k_rem0, lane_reduce=True)
    s_above0 = suf_at(dstar0 + 1)
    s_cand0 = suf_at(dstar0)
    prefix1 = lax.shift_left(dstar0, _i32(20))
    k_rem1 = k_rem0 - s_above0

    def fine_level(shift, pref, k_rem, read_fn, n_iters, dedup):
        pmask = _i32((-1 << (shift + 10)) & 0x3FFFFFFF)
        zero_hist(_NBINS if dedup else _NBINS * 16)

        def hb(i, _):
            bits = read_fn(i)[1]
            m = (bits >= 0) & ((bits & pmask) == pref)
            digit = lax.shift_right_logical(bits, shift) & (_NBINS - 1)
            if dedup:
                dup, last = plsc.scan_count(digit, mask=m)
                plsc.addupdate_scatter(hist_v, [digit], dup, mask=last & m)
            else:
                plsc.addupdate_scatter(hist_v, [hist_lane_base + digit],
                                       ones, mask=m)
            return 0
        lax.fori_loop(0, n_iters, hb, 0)
        dstar = suffix_pick(k_rem, lane_reduce=not dedup)
        s_next = suf_at(dstar + 1)
        return pref | lax.shift_left(dstar, _i32(shift)), k_rem - s_next

    def selection(v300, t_ties, read_fn, n_iters):
        def sel(i, carry):
            gt_off, eq_taken = carry
            idxv, bits = read_fn(i)
            valid = bits >= 0
            isgt = valid & (bits > v300)
            iseq = valid & (bits == v300)
            plsc.store_compressed(gt_v.at[pl.ds(gt_off, 16)], idxv, mask=isgt)
            gt_off = gt_off + popcnt(isgt)
            eqrank = plsc.cumsum(iseq.astype(_i32))
            take = iseq & ((eq_taken + eqrank) <= t_ties)
            plsc.store_compressed(eq_v.at[pl.ds(eq_taken, 16)], idxv,
                                  mask=take)
            eq_taken = eq_taken + popcnt(take)
            return (gt_off, eq_taken)
        lax.fori_loop(0, n_iters, sel, (_i32(0), _i32(0)))

    def read_full(i):
        v = probs_v[pl.ds(i * 16, 16)]
        return i * 16 + lanes, lax.bitcast_convert_type(v, _i32)

    def read_cand(i):
        ci = cand_v[pl.ds(i * 16, 16)]
        v = plsc.load_gather(probs_v, [ci])
        return ci, lax.bitcast_convert_type(v, _i32)

    @pl.when(s_cand0 <= _CAP)
    def _fast():
        def cb(i, off):
            for u in range(4):
                idxv, bits = read_full(i * 4 + u)
                m = (bits >= 0) & (bits >= prefix1)
                plsc.store_compressed(cand_v.at[pl.ds(off, 16)], idxv, mask=m)
                off = off + popcnt(m)
            return off
        lax.fori_loop(0, _NV // 4, cb, _i32(0))
        cand_v[pl.ds(s_cand0, 16)] = jnp.full((16,), _PAD_IDX, _i32)
        nct = (s_cand0 + 15) // 16
        pref, krem = fine_level(10, prefix1, k_rem1, read_cand, nct, True)
        pref, krem = fine_level(0, pref, krem, read_cand, nct, True)
        selection(pref, krem, read_cand, nct)
        tsm[0] = krem

    @pl.when(s_cand0 > _CAP)
    def _slow():
        pref, krem = fine_level(10, prefix1, k_rem1, read_full, _NV, False)
        pref, krem = fine_level(0, pref, krem, read_full, _NV, False)
        selection(pref, krem, read_full, _NV)
        tsm[0] = krem

    t_ties = tsm[0]
    g_cnt = _i32(_K) - t_ties

    def merge_body(j, _):
        ev = eq_v[pl.ds(j * 16, 16)]
        m = (j * 16 + lanes) < t_ties
        plsc.store_compressed(gt_v.at[pl.ds(g_cnt + j * 16, 16)], ev, mask=m)
        return 0
    lax.fori_loop(0, (t_ties + 15) // 16, merge_body, 0)

    gt_v[pl.ds(_K, 16)] = jnp.full((16,), _PAD_IDX, _i32)

    for j in range(_KV):
        si = gt_v[pl.ds(j * 16, 16)]
        val_v[pl.ds(j * 16, 16)] = plsc.load_gather(probs_v, [si])

    def key_body(j, _):
        bits = lax.bitcast_convert_type(val_v[pl.ds(j * 16, 16)], _i32)
        key = jnp.where(bits >= 0, _i32(0x40000000) - bits, _i32(0x50000000))
        key_v[pl.ds(j * 16, 16)] = key
        pay_v[pl.ds(j * 16, 16)] = j * 16 + lanes
        return 0
    lax.fori_loop(0, _KV, key_body, 0)

    bufs = ((key_v, pay_v), (key_a, pay_a))
    for p in range(7):
        kc, pc = bufs[p % 2]
        ka, pa = bufs[1 - p % 2]
        shift = 5 * p
        hist_v[pl.ds(0, 16)] = jnp.zeros((16,), _i32)
        hist_v[pl.ds(16, 16)] = jnp.zeros((16,), _i32)

        def rh(j, _, kc=kc, shift=shift):
            d = lax.shift_right_logical(kc[pl.ds(j * 16, 16)], shift) & 31
            dup, last = plsc.scan_count(d)
            plsc.addupdate_scatter(hist_v, [d], dup, mask=last)
            return 0
        lax.fori_loop(0, _KV, rh, 0)

        h0 = hist_v[pl.ds(0, 16)]
        h1 = hist_v[pl.ds(16, 16)]
        c0 = plsc.cumsum(h0)
        c1 = plsc.cumsum(h1)
        suf_v[pl.ds(0, 16)] = c0 - h0
        suf_v[pl.ds(16, 16)] = c1 - h1 + jnp.max(c0)

        def rp(j, _, kc=kc, pc=pc, ka=ka, pa=pa, shift=shift):
            k = kc[pl.ds(j * 16, 16)]
            pay = pc[pl.ds(j * 16, 16)]
            d = lax.shift_right_logical(k, shift) & 31
            base = plsc.load_gather(suf_v, [d])
            dup, last = plsc.scan_count(d)
            pos = base + dup - 1
            plsc.store_scatter(ka, [pos], k)
            plsc.store_scatter(pa, [pos], pay)
            plsc.addupdate_scatter(suf_v, [d], dup, mask=last)
            return 0
        lax.fori_loop(0, _KV, rp, 0)
    pay_fin = bufs[7 % 2][1]

    w_scale = plsc.load_gather(ts_v, [jnp.full((16,), 1, _i32)])
    h_scale = plsc.load_gather(ts_v, [jnp.full((16,), 2, _i32)])

    def out_body(iv, _):
        p = pay_fin[pl.ds(iv * 16, 16)]
        vi = plsc.load_gather(val_v, [p])
        sidx = plsc.load_gather(gt_v, [p])
        qq = sidx // _C
        cc = sidx - qq * _C
        outs_v[pl.ds(iv * 16, 16)] = vi
        outl_v[pl.ds(iv * 16, 16)] = cc
        b0 = qq * 4
        cx = plsc.load_gather(boxes_v, [b0])
        cy = plsc.load_gather(boxes_v, [b0 + 1])
        hw = 0.5 * plsc.load_gather(boxes_v, [b0 + 2])
        hh = 0.5 * plsc.load_gather(boxes_v, [b0 + 3])
        rb = (iv * 16 + lanes) * 4
        plsc.store_scatter(outb_v, [rb], (cx - hw) * w_scale)
        plsc.store_scatter(outb_v, [rb + 1], (cy - hh) * h_scale)
        plsc.store_scatter(outb_v, [rb + 2], (cx + hw) * w_scale)
        plsc.store_scatter(outb_v, [rb + 3], (cy + hh) * h_scale)
        return 0
    lax.fori_loop(0, _KV, out_body, 0)

    pltpu.sync_copy(outs_v, sc_out.at[wid])
    pltpu.sync_copy(outl_v, lb_out.at[wid])
    pltpu.sync_copy(outb_v, bx_out.at[wid])


@jax.jit
def kernel(pred_logits, pred_boxes, target_sizes):
    probs = jax.nn.sigmoid(pred_logits).reshape(_B, _N)
    probs_p = jnp.concatenate(
        [probs, jnp.full((_B, _NPAD - _N), -1.0, _f32)], axis=1)
    boxes = jnp.concatenate(
        [pred_boxes.reshape(_B, _Q * 4).astype(_f32),
         jnp.zeros((_B, _BOXPAD - _Q * 4), _f32)], axis=1)
    ts = target_sizes.astype(_f32)
    ts_p = jnp.concatenate(
        [jnp.zeros((_B, 1), _f32), ts[:, 1:2], ts[:, 0:1],
         jnp.zeros((_B, 125), _f32)], axis=1)

    mesh = plsc.VectorSubcoreMesh(core_axis_name="c", subcore_axis_name="s",
                                  num_cores=2, num_subcores=16)
    fn = pl.kernel(
        _sc_body,
        out_type=(
            jax.ShapeDtypeStruct((_B, _OUTP), _f32),
            jax.ShapeDtypeStruct((_B, _OUTP), _i32),
            jax.ShapeDtypeStruct((_B, _OUTB), _f32),
        ),
        mesh=mesh,
        compiler_params=pltpu.CompilerParams(needs_layout_passes=False),
        scratch_types=[
            pltpu.VMEM((_NPAD,), _f32),
            pltpu.VMEM((_BOXPAD,), _f32),
            pltpu.VMEM((128,), _f32),
            pltpu.VMEM((_NBINS * 16,), _i32),
            pltpu.VMEM((_NBINS + 16,), _i32),
            pltpu.VMEM((_CAP + 32,), _i32),
            pltpu.VMEM((320,), _i32),
            pltpu.VMEM((320,), _i32),
            pltpu.VMEM((_KV * 16,), _f32),
            pltpu.VMEM((_KV * 16,), _i32),
            pltpu.VMEM((_KV * 16,), _i32),
            pltpu.VMEM((_KV * 16,), _i32),
            pltpu.VMEM((_KV * 16,), _i32),
            pltpu.VMEM((_OUTP,), _f32),
            pltpu.VMEM((_OUTP,), _i32),
            pltpu.VMEM((_OUTB,), _f32),
            pltpu.SMEM((8,), _i32),
        ],
    )
    s, l, b = fn(probs_p, boxes, ts_p)
    return (s[:, :_K], l[:, :_K],
            b[:, :_K * 4].reshape(_B, _K, 4))


# speedup vs baseline: 1.1392x; 1.1392x over previous
import functools

import jax
import jax.numpy as jnp
from jax import lax
from jax.experimental import pallas as pl
from jax.experimental.pallas import tpu as pltpu
from jax.experimental.pallas import tpu_sc as plsc

_B, _Q, _C = 32, 900, 91
_N = _Q * _C
_NPAD = 81920
_NV = _NPAD // 16
_K = 300
_KV = 19
_OUTP = 384
_OUTB = 1280
_NBINS = 1024
_PAD_IDX = _N + 4
_BOXPAD = 3712
_CAP = 4096

_i32 = jnp.int32
_f32 = jnp.float32


def _sc_body(probs_hbm, boxes_hbm, ts_hbm, sc_out, lb_out, bx_out,
             probs_v, boxes_v, ts_v, hist_v, suf_v, cand_v, gt_v, eq_v,
             val_v, key_v, key_a, pay_v, pay_a, outs_v, outl_v, outb_v, tsm):
    lanes = lax.broadcasted_iota(_i32, (16,), 0)
    wid = lax.axis_index("s") * 2 + lax.axis_index("c")

    pltpu.sync_copy(probs_hbm.at[wid], probs_v)
    pltpu.sync_copy(boxes_hbm.at[wid], boxes_v)
    pltpu.sync_copy(ts_hbm.at[wid], ts_v)

    suf_v[pl.ds(_NBINS, 16)] = jnp.zeros((16,), _i32)

    hist_lane_base = lanes * _NBINS
    ones = jnp.ones((16,), _i32)

    def popcnt(m):
        return plsc.all_reduce_population_count(m)[0]

    def zero_hist(nwords):
        def zb(i, _):
            hist_v[pl.ds(i * 16, 16)] = jnp.zeros((16,), _i32)
            return 0
        lax.fori_loop(0, nwords // 16, zb, 0)

    def suffix_pick(k_rem, lane_reduce):
        def sb(g_iter, carry):
            s_carry, dstar = carry
            g = 63 - g_iter
            if lane_reduce:
                tot = jnp.zeros((16,), _i32)
                for l in range(16):
                    tot = tot + hist_v[pl.ds(l * _NBINS + g * 16, 16)]
            else:
                tot = hist_v[pl.ds(g * 16, 16)]
            rcs = jnp.flip(plsc.cumsum(jnp.flip(tot))) + s_carry
            suf_v[pl.ds(g * 16, 16)] = rcs
            dvec = g * 16 + lanes
            cand = jnp.max(jnp.where(rcs >= k_rem, dvec, -1))
            return (jnp.max(rcs), jnp.maximum(dstar, cand))
        _, dstar = lax.fori_loop(0, 64, sb, (_i32(0), _i32(-1)))
        return dstar

    def suf_at(d):
        return jnp.max(plsc.load_gather(suf_v, [jnp.full((16,), 0, _i32) + d]))

    zero_hist(_NBINS)

    def hists_body(i, _):
        for u in range(4):
            v = probs_v[pl.ds((i * 4 + u) * 256, 16)]
            bits = lax.bitcast_convert_type(v, _i32)
            m = bits >= 0
            digit = lax.shift_right_logical(bits, 20) & (_NBINS - 1)
            dup, last = plsc.scan_count(digit, mask=m)
            plsc.addupdate_scatter(hist_v, [digit], dup, mask=last & m)
        return 0
    lax.fori_loop(0, _NV // 64, hists_body, 0)

    dstar_s = suffix_pick(_i32(38), lane_reduce=False)
    prefix1 = lax.shift_left(dstar_s, _i32(20))

    def fine_level(shift, pref, k_rem, read_fn, n_iters, dedup):
        pmask = _i32((-1 << (shift + 10)) & 0x3FFFFFFF)
        zero_hist(_NBINS if dedup else _NBINS * 16)

        def hb(i, _):
            bits = read_fn(i)[1]
            m = (bits >= 0) & ((bits & pmask) == pref)
            digit = lax.shift_right_logical(bits, shift) & (_NBINS - 1)
            if dedup:
                dup, last = plsc.scan_count(digit, mask=m)
                plsc.addupdate_scatter(hist_v, [digit], dup, mask=last & m)
            else:
                plsc.addupdate_scatter(hist_v, [hist_lane_base + digit],
                                       ones, mask=m)
            return 0
        lax.fori_loop(0, n_iters, hb, 0)
        dstar = suffix_pick(k_rem, lane_reduce=not dedup)
        s_next = suf_at(dstar + 1)
        return pref | lax.shift_left(dstar, _i32(shift)), k_rem - s_next

    def selection(v300, t_ties, read_fn, n_iters):
        def sel(i, carry):
            gt_off, eq_taken = carry
            idxv, bits = read_fn(i)
            valid = bits >= 0
            isgt = valid & (bits > v300)
            iseq = valid & (bits == v300)
            plsc.store_compressed(gt_v.at[pl.ds(gt_off, 16)], idxv, mask=isgt)
            gt_off = gt_off + popcnt(isgt)
            eqrank = plsc.cumsum(iseq.astype(_i32))
            take = iseq & ((eq_taken + eqrank) <= t_ties)
            plsc.store_compressed(eq_v.at[pl.ds(eq_taken, 16)], idxv,
                                  mask=take)
            eq_taken = eq_taken + popcnt(take)
            return (gt_off, eq_taken)
        lax.fori_loop(0, n_iters, sel, (_i32(0), _i32(0)))

    def read_full(i):
        v = probs_v[pl.ds(i * 16, 16)]
        return i * 16 + lanes, lax.bitcast_convert_type(v, _i32)

    def read_cand(i):
        ci = cand_v[pl.ds(i * 16, 16)]
        v = plsc.load_gather(probs_v, [ci])
        return ci, lax.bitcast_convert_type(v, _i32)

    def cb(i, off):
        for u in range(4):
            idxv, bits = read_full(i * 4 + u)
            m = (bits >= 0) & (bits >= prefix1)
            plsc.store_compressed(cand_v.at[pl.ds(jnp.minimum(off, _CAP), 16)],
                                  idxv, mask=m & (off <= _CAP))
            off = off + popcnt(m)
        return off
    s_cand = lax.fori_loop(0, _NV // 4, cb, _i32(0))

    @pl.when((s_cand >= _K) & (s_cand <= _CAP))
    def _fast():
        cand_v[pl.ds(s_cand, 16)] = jnp.full((16,), _PAD_IDX, _i32)
        nct = (s_cand + 15) // 16
        pref, krem = fine_level(20, _i32(0), _i32(_K), read_cand, nct, True)
        pref, krem = fine_level(10, pref, krem, read_cand, nct, True)
        pref, krem = fine_level(0, pref, krem, read_cand, nct, True)
        selection(pref, krem, read_cand, nct)
        tsm[0] = krem

    @pl.when((s_cand < _K) | (s_cand > _CAP))
    def _slow():
        zero_hist(_NBINS * 16)

        def h0(i, _):
            for u in range(4):
                _, bits = read_full(i * 4 + u)
                m = bits >= 0
                digit = lax.shift_right_logical(bits, 20) & (_NBINS - 1)
                plsc.addupdate_scatter(hist_v, [hist_lane_base + digit],
                                       ones, mask=m)
            return 0
        lax.fori_loop(0, _NV // 4, h0, 0)
        dstar0 = suffix_pick(_i32(_K), lane_reduce=True)
        pref = lax.shift_left(dstar0, _i32(20))
        krem = _i32(_K) - suf_at(dstar0 + 1)
        pref, krem = fine_level(10, pref, krem, read_full, _NV, False)
        pref, krem = fine_level(0, pref, krem, read_full, _NV, False)
        selection(pref, krem, read_full, _NV)
        tsm[0] = krem

    t_ties = tsm[0]
    g_cnt = _i32(_K) - t_ties

    def merge_body(j, _):
        ev = eq_v[pl.ds(j * 16, 16)]
        m = (j * 16 + lanes) < t_ties
        plsc.store_compressed(gt_v.at[pl.ds(g_cnt + j * 16, 16)], ev, mask=m)
        return 0
    lax.fori_loop(0, (t_ties + 15) // 16, merge_body, 0)

    gt_v[pl.ds(_K, 16)] = jnp.full((16,), _PAD_IDX, _i32)

    for j in range(_KV):
        si = gt_v[pl.ds(j * 16, 16)]
        val_v[pl.ds(j * 16, 16)] = plsc.load_gather(probs_v, [si])

    def key_body(j, _):
        bits = lax.bitcast_convert_type(val_v[pl.ds(j * 16, 16)], _i32)
        key = jnp.where(bits >= 0, _i32(0x40000000) - bits, _i32(0x50000000))
        key_v[pl.ds(j * 16, 16)] = key
        pay_v[pl.ds(j * 16, 16)] = j * 16 + lanes
        return 0
    lax.fori_loop(0, _KV, key_body, 0)

    bufs = ((key_v, pay_v), (key_a, pay_a))
    for p in range(7):
        kc, pc = bufs[p % 2]
        ka, pa = bufs[1 - p % 2]
        shift = 5 * p
        hist_v[pl.ds(0, 16)] = jnp.zeros((16,), _i32)
        hist_v[pl.ds(16, 16)] = jnp.zeros((16,), _i32)

        def rh(j, _, kc=kc, shift=shift):
            d = lax.shift_right_logical(kc[pl.ds(j * 16, 16)], shift) & 31
            dup, last = plsc.scan_count(d)
            plsc.addupdate_scatter(hist_v, [d], dup, mask=last)
            return 0
        lax.fori_loop(0, _KV, rh, 0)

        h0 = hist_v[pl.ds(0, 16)]
        h1 = hist_v[pl.ds(16, 16)]
        c0 = plsc.cumsum(h0)
        c1 = plsc.cumsum(h1)
        suf_v[pl.ds(0, 16)] = c0 - h0
        suf_v[pl.ds(16, 16)] = c1 - h1 + jnp.max(c0)

        def rp(j, _, kc=kc, pc=pc, ka=ka, pa=pa, shift=shift):
            k = kc[pl.ds(j * 16, 16)]
            pay = pc[pl.ds(j * 16, 16)]
            d = lax.shift_right_logical(k, shift) & 31
            base = plsc.load_gather(suf_v, [d])
            dup, last = plsc.scan_count(d)
            pos = base + dup - 1
            plsc.store_scatter(ka, [pos], k)
            plsc.store_scatter(pa, [pos], pay)
            plsc.addupdate_scatter(suf_v, [d], dup, mask=last)
            return 0
        lax.fori_loop(0, _KV, rp, 0)
    pay_fin = bufs[7 % 2][1]

    w_scale = plsc.load_gather(ts_v, [jnp.full((16,), 1, _i32)])
    h_scale = plsc.load_gather(ts_v, [jnp.full((16,), 2, _i32)])

    def out_body(iv, _):
        p = pay_fin[pl.ds(iv * 16, 16)]
        vi = plsc.load_gather(val_v, [p])
        sidx = plsc.load_gather(gt_v, [p])
        qq = sidx // _C
        cc = sidx - qq * _C
        outs_v[pl.ds(iv * 16, 16)] = vi
        outl_v[pl.ds(iv * 16, 16)] = cc
        b0 = qq * 4
        cx = plsc.load_gather(boxes_v, [b0])
        cy = plsc.load_gather(boxes_v, [b0 + 1])
        hw = 0.5 * plsc.load_gather(boxes_v, [b0 + 2])
        hh = 0.5 * plsc.load_gather(boxes_v, [b0 + 3])
        rb = (iv * 16 + lanes) * 4
        plsc.store_scatter(outb_v, [rb], (cx - hw) * w_scale)
        plsc.store_scatter(outb_v, [rb + 1], (cy - hh) * h_scale)
        plsc.store_scatter(outb_v, [rb + 2], (cx + hw) * w_scale)
        plsc.store_scatter(outb_v, [rb + 3], (cy + hh) * h_scale)
        return 0
    lax.fori_loop(0, _KV, out_body, 0)

    pltpu.sync_copy(outs_v, sc_out.at[wid])
    pltpu.sync_copy(outl_v, lb_out.at[wid])
    pltpu.sync_copy(outb_v, bx_out.at[wid])


@jax.jit
def kernel(pred_logits, pred_boxes, target_sizes):
    probs = jax.nn.sigmoid(pred_logits).reshape(_B, _N)
    probs_p = jnp.concatenate(
        [probs, jnp.full((_B, _NPAD - _N), -1.0, _f32)], axis=1)
    boxes = jnp.concatenate(
        [pred_boxes.reshape(_B, _Q * 4).astype(_f32),
         jnp.zeros((_B, _BOXPAD - _Q * 4), _f32)], axis=1)
    ts = target_sizes.astype(_f32)
    ts_p = jnp.concatenate(
        [jnp.zeros((_B, 1), _f32), ts[:, 1:2], ts[:, 0:1],
         jnp.zeros((_B, 125), _f32)], axis=1)

    mesh = plsc.VectorSubcoreMesh(core_axis_name="c", subcore_axis_name="s",
                                  num_cores=2, num_subcores=16)
    fn = pl.kernel(
        _sc_body,
        out_type=(
            jax.ShapeDtypeStruct((_B, _OUTP), _f32),
            jax.ShapeDtypeStruct((_B, _OUTP), _i32),
            jax.ShapeDtypeStruct((_B, _OUTB), _f32),
        ),
        mesh=mesh,
        compiler_params=pltpu.CompilerParams(needs_layout_passes=False),
        scratch_types=[
            pltpu.VMEM((_NPAD,), _f32),
            pltpu.VMEM((_BOXPAD,), _f32),
            pltpu.VMEM((128,), _f32),
            pltpu.VMEM((_NBINS * 16,), _i32),
            pltpu.VMEM((_NBINS + 16,), _i32),
            pltpu.VMEM((_CAP + 32,), _i32),
            pltpu.VMEM((320,), _i32),
            pltpu.VMEM((320,), _i32),
            pltpu.VMEM((_KV * 16,), _f32),
            pltpu.VMEM((_KV * 16,), _i32),
            pltpu.VMEM((_KV * 16,), _i32),
            pltpu.VMEM((_KV * 16,), _i32),
            pltpu.VMEM((_KV * 16,), _i32),
            pltpu.VMEM((_OUTP,), _f32),
            pltpu.VMEM((_OUTP,), _i32),
            pltpu.VMEM((_OUTB,), _f32),
            pltpu.SMEM((8,), _i32),
        ],
    )
    s, l, b = fn(probs_p, boxes, ts_p)
    return (s[:, :_K], l[:, :_K],
            b[:, :_K * 4].reshape(_B, _K, 4))


# speedup vs baseline: 1.1400x; 1.0007x over previous
import functools

import jax
import jax.numpy as jnp
from jax import lax
from jax.experimental import pallas as pl
from jax.experimental.pallas import tpu as pltpu
from jax.experimental.pallas import tpu_sc as plsc

_B, _Q, _C = 32, 900, 91
_N = _Q * _C
_NPAD = 81920
_NV = _NPAD // 16
_K = 300
_KV = 19
_OUTP = 384
_OUTB = 1280
_NBINS = 1024
_PAD_IDX = _N + 4
_BOXPAD = 3712
_CAP = 4096

_i32 = jnp.int32
_f32 = jnp.float32


def _sc_body(probs_hbm, boxes_hbm, ts_hbm, sc_out, lb_out, bx_out,
             probs_v, boxes_v, ts_v, hist_v, suf_v, cand_v, gt_v, eq_v,
             val_v, key_v, key_a, pay_v, pay_a, outs_v, outl_v, outb_v, tsm):
    lanes = lax.broadcasted_iota(_i32, (16,), 0)
    wid = lax.axis_index("s") * 2 + lax.axis_index("c")

    pltpu.sync_copy(probs_hbm.at[wid], probs_v)
    pltpu.sync_copy(boxes_hbm.at[wid], boxes_v)
    pltpu.sync_copy(ts_hbm.at[wid], ts_v)

    suf_v[pl.ds(_NBINS, 16)] = jnp.zeros((16,), _i32)

    hist_lane_base = lanes * _NBINS
    ones = jnp.ones((16,), _i32)

    def popcnt(m):
        return plsc.all_reduce_population_count(m)[0]

    def zero_hist(nwords):
        def zb(i, _):
            hist_v[pl.ds(i * 16, 16)] = jnp.zeros((16,), _i32)
            return 0
        lax.fori_loop(0, nwords // 16, zb, 0)

    def suffix_pick(k_rem, lane_reduce):
        def sb(g_iter, carry):
            s_carry, dstar = carry
            g = 63 - g_iter
            if lane_reduce:
                tot = jnp.zeros((16,), _i32)
                for l in range(16):
                    tot = tot + hist_v[pl.ds(l * _NBINS + g * 16, 16)]
            else:
                tot = hist_v[pl.ds(g * 16, 16)]
            rcs = jnp.flip(plsc.cumsum(jnp.flip(tot))) + s_carry
            suf_v[pl.ds(g * 16, 16)] = rcs
            dvec = g * 16 + lanes
            cand = jnp.max(jnp.where(rcs >= k_rem, dvec, -1))
            return (jnp.max(rcs), jnp.maximum(dstar, cand))
        _, dstar = lax.fori_loop(0, 64, sb, (_i32(0), _i32(-1)))
        return dstar

    def suf_at(d):
        return jnp.max(plsc.load_gather(suf_v, [jnp.full((16,), 0, _i32) + d]))

    zero_hist(_NBINS)

    def hists_body(i, _):
        for u in range(4):
            v = probs_v[pl.ds((i * 4 + u) * 256, 16)]
            bits = lax.bitcast_convert_type(v, _i32)
            m = bits >= 0
            digit = lax.shift_right_logical(bits, 20) & (_NBINS - 1)
            dup, last = plsc.scan_count(digit, mask=m)
            plsc.addupdate_scatter(hist_v, [digit], dup, mask=last & m)
        return 0
    lax.fori_loop(0, _NV // 64, hists_body, 0)

    dstar_s = suffix_pick(_i32(38), lane_reduce=False)
    prefix1 = lax.shift_left(dstar_s, _i32(20))

    def fine_level(shift, pref, k_rem, read_fn, n_iters, dedup):
        pmask = _i32((-1 << (shift + 10)) & 0x3FFFFFFF)
        zero_hist(_NBINS if dedup else _NBINS * 16)

        def hb(i, _):
            bits = read_fn(i)[1]
            m = (bits >= 0) & ((bits & pmask) == pref)
            digit = lax.shift_right_logical(bits, shift) & (_NBINS - 1)
            if dedup:
                dup, last = plsc.scan_count(digit, mask=m)
                plsc.addupdate_scatter(hist_v, [digit], dup, mask=last & m)
            else:
                plsc.addupdate_scatter(hist_v, [hist_lane_base + digit],
                                       ones, mask=m)
            return 0
        lax.fori_loop(0, n_iters, hb, 0)
        dstar = suffix_pick(k_rem, lane_reduce=not dedup)
        s_next = suf_at(dstar + 1)
        return pref | lax.shift_left(dstar, _i32(shift)), k_rem - s_next

    def selection(v300, t_ties, read_fn, n_iters):
        def sel(i, carry):
            gt_off, eq_taken = carry
            idxv, bits = read_fn(i)
            valid = bits >= 0
            isgt = valid & (bits > v300)
            iseq = valid & (bits == v300)
            plsc.store_compressed(gt_v.at[pl.ds(gt_off, 16)], idxv, mask=isgt)
            gt_off = gt_off + popcnt(isgt)
            eqrank = plsc.cumsum(iseq.astype(_i32))
            take = iseq & ((eq_taken + eqrank) <= t_ties)
            plsc.store_compressed(eq_v.at[pl.ds(eq_taken, 16)], idxv,
                                  mask=take)
            eq_taken = eq_taken + popcnt(take)
            return (gt_off, eq_taken)
        lax.fori_loop(0, n_iters, sel, (_i32(0), _i32(0)))

    def read_full(i):
        v = probs_v[pl.ds(i * 16, 16)]
        return i * 16 + lanes, lax.bitcast_convert_type(v, _i32)

    def read_cand(i):
        ci = cand_v[pl.ds(i * 16, 16)]
        v = plsc.load_gather(probs_v, [ci])
        return ci, lax.bitcast_convert_type(v, _i32)

    def cb(i, off):
        for u in range(4):
            idxv, bits = read_full(i * 4 + u)
            m = (bits >= 0) & (bits >= prefix1)
            plsc.store_compressed(cand_v.at[pl.ds(jnp.minimum(off, _CAP), 16)],
                                  idxv, mask=m & (off <= _CAP))
            off = off + popcnt(m)
        return off
    s_cand = lax.fori_loop(0, _NV // 4, cb, _i32(0))

    @pl.when((s_cand >= _K) & (s_cand <= _CAP))
    def _fast():
        cand_v[pl.ds(s_cand, 16)] = jnp.full((16,), _PAD_IDX, _i32)
        nct = (s_cand + 15) // 16
        pref, krem = fine_level(20, _i32(0), _i32(_K), read_cand, nct, True)
        pref, krem = fine_level(10, pref, krem, read_cand, nct, True)
        pref, krem = fine_level(0, pref, krem, read_cand, nct, True)
        selection(pref, krem, read_cand, nct)
        tsm[0] = krem

    @pl.when((s_cand < _K) | (s_cand > _CAP))
    def _slow():
        zero_hist(_NBINS * 16)

        def h0(i, _):
            for u in range(4):
                _, bits = read_full(i * 4 + u)
                m = bits >= 0
                digit = lax.shift_right_logical(bits, 20) & (_NBINS - 1)
                plsc.addupdate_scatter(hist_v, [hist_lane_base + digit],
                                       ones, mask=m)
            return 0
        lax.fori_loop(0, _NV // 4, h0, 0)
        dstar0 = suffix_pick(_i32(_K), lane_reduce=True)
        pref = lax.shift_left(dstar0, _i32(20))
        krem = _i32(_K) - suf_at(dstar0 + 1)
        pref, krem = fine_level(10, pref, krem, read_full, _NV, False)
        pref, krem = fine_level(0, pref, krem, read_full, _NV, False)
        selection(pref, krem, read_full, _NV)
        tsm[0] = krem

    t_ties = tsm[0]
    g_cnt = _i32(_K) - t_ties

    def merge_body(j, _):
        ev = eq_v[pl.ds(j * 16, 16)]
        m = (j * 16 + lanes) < t_ties
        plsc.store_compressed(gt_v.at[pl.ds(g_cnt + j * 16, 16)], ev, mask=m)
        return 0
    lax.fori_loop(0, (t_ties + 15) // 16, merge_body, 0)

    gt_v[pl.ds(_K, 16)] = jnp.full((16,), _PAD_IDX, _i32)

    for j in range(_KV):
        si = gt_v[pl.ds(j * 16, 16)]
        val_v[pl.ds(j * 16, 16)] = plsc.load_gather(probs_v, [si])

    def key_body(j, _):
        bits = lax.bitcast_convert_type(val_v[pl.ds(j * 16, 16)], _i32)
        key = jnp.where(bits >= 0, _i32(0x40000000) - bits, _i32(0x50000000))
        key_v[pl.ds(j * 16, 16)] = key
        pay_v[pl.ds(j * 16, 16)] = j * 16 + lanes
        return 0
    lax.fori_loop(0, _KV, key_body, 0)

    bufs = ((key_v, pay_v), (key_a, pay_a))
    for p in range(7):
        kc, pc = bufs[p % 2]
        ka, pa = bufs[1 - p % 2]
        shift = 5 * p
        hist_v[pl.ds(0, 16)] = jnp.zeros((16,), _i32)
        hist_v[pl.ds(16, 16)] = jnp.zeros((16,), _i32)

        def rh(j, _, kc=kc, shift=shift):
            d = lax.shift_right_logical(kc[pl.ds(j * 16, 16)], shift) & 31
            dup, last = plsc.scan_count(d)
            plsc.addupdate_scatter(hist_v, [d], dup, mask=last)
            return 0
        lax.fori_loop(0, _KV, rh, 0)

        h0 = hist_v[pl.ds(0, 16)]
        h1 = hist_v[pl.ds(16, 16)]
        c0 = plsc.cumsum(h0)
        c1 = plsc.cumsum(h1)
        suf_v[pl.ds(0, 16)] = c0 - h0
        suf_v[pl.ds(16, 16)] = c1 - h1 + jnp.max(c0)

        def rp(j, _, kc=kc, pc=pc, ka=ka, pa=pa, shift=shift):
            k = kc[pl.ds(j * 16, 16)]
            pay = pc[pl.ds(j * 16, 16)]
            d = lax.shift_right_logical(k, shift) & 31
            base = plsc.load_gather(suf_v, [d])
            dup, last = plsc.scan_count(d)
            pos = base + dup - 1
            plsc.store_scatter(ka, [pos], k)
            plsc.store_scatter(pa, [pos], pay)
            plsc.addupdate_scatter(suf_v, [d], dup, mask=last)
            return 0
        lax.fori_loop(0, _KV, rp, 0)
    pay_fin = bufs[7 % 2][1]

    w_scale = plsc.load_gather(ts_v, [jnp.full((16,), 1, _i32)])
    h_scale = plsc.load_gather(ts_v, [jnp.full((16,), 2, _i32)])

    def out_body(iv, _):
        p = pay_fin[pl.ds(iv * 16, 16)]
        vi = plsc.load_gather(val_v, [p])
        sidx = plsc.load_gather(gt_v, [p])
        qq = sidx // _C
        cc = sidx - qq * _C
        outs_v[pl.ds(iv * 16, 16)] = vi
        outl_v[pl.ds(iv * 16, 16)] = cc
        b0 = qq * 4
        cx = plsc.load_gather(boxes_v, [b0])
        cy = plsc.load_gather(boxes_v, [b0 + 1])
        hw = 0.5 * plsc.load_gather(boxes_v, [b0 + 2])
        hh = 0.5 * plsc.load_gather(boxes_v, [b0 + 3])
        rb = (iv * 16 + lanes) * 4
        plsc.store_scatter(outb_v, [rb], (cx - hw) * w_scale)
        plsc.store_scatter(outb_v, [rb + 1], (cy - hh) * h_scale)
        plsc.store_scatter(outb_v, [rb + 2], (cx + hw) * w_scale)
        plsc.store_scatter(outb_v, [rb + 3], (cy + hh) * h_scale)
        return 0
    lax.fori_loop(0, _KV, out_body, 0)

    pltpu.sync_copy(outs_v, sc_out.at[wid])
    pltpu.sync_copy(outl_v, lb_out.at[wid])
    pltpu.sync_copy(outb_v, bx_out.at[wid])


@jax.jit
def kernel(pred_logits, pred_boxes, target_sizes):
    probs_p = jnp.pad(jax.nn.sigmoid(pred_logits).reshape(_B, _N),
                      ((0, 0), (0, _NPAD - _N)), constant_values=-1.0)
    boxes = jnp.pad(pred_boxes.reshape(_B, _Q * 4).astype(_f32),
                    ((0, 0), (0, _BOXPAD - _Q * 4)))
    ts = target_sizes.astype(_f32)
    ts_p = jnp.concatenate(
        [jnp.zeros((_B, 1), _f32), ts[:, 1:2], ts[:, 0:1],
         jnp.zeros((_B, 125), _f32)], axis=1)

    mesh = plsc.VectorSubcoreMesh(core_axis_name="c", subcore_axis_name="s",
                                  num_cores=2, num_subcores=16)
    fn = pl.kernel(
        _sc_body,
        out_type=(
            jax.ShapeDtypeStruct((_B, _OUTP), _f32),
            jax.ShapeDtypeStruct((_B, _OUTP), _i32),
            jax.ShapeDtypeStruct((_B, _OUTB), _f32),
        ),
        mesh=mesh,
        compiler_params=pltpu.CompilerParams(needs_layout_passes=False),
        scratch_types=[
            pltpu.VMEM((_NPAD,), _f32),
            pltpu.VMEM((_BOXPAD,), _f32),
            pltpu.VMEM((128,), _f32),
            pltpu.VMEM((_NBINS * 16,), _i32),
            pltpu.VMEM((_NBINS + 16,), _i32),
            pltpu.VMEM((_CAP + 32,), _i32),
            pltpu.VMEM((320,), _i32),
            pltpu.VMEM((320,), _i32),
            pltpu.VMEM((_KV * 16,), _f32),
            pltpu.VMEM((_KV * 16,), _i32),
            pltpu.VMEM((_KV * 16,), _i32),
            pltpu.VMEM((_KV * 16,), _i32),
            pltpu.VMEM((_KV * 16,), _i32),
            pltpu.VMEM((_OUTP,), _f32),
            pltpu.VMEM((_OUTP,), _i32),
            pltpu.VMEM((_OUTB,), _f32),
            pltpu.SMEM((8,), _i32),
        ],
    )
    s, l, b = fn(probs_p, boxes, ts_p)
    return (s[:, :_K], l[:, :_K],
            b[:, :_K * 4].reshape(_B, _K, 4))
